# baseline (speedup 1.0000x reference)
import functools

import jax
import jax.numpy as jnp
from jax import lax
from jax.experimental import pallas as pl
from jax.experimental.pallas import tpu as pltpu
from jax.experimental.pallas import tpu_sc as plsc

_NW = 32


@functools.partial(jax.jit, static_argnames=("n", "q"))
def _sc_gather(flat_ids, W, n, q):
    per_w = n // _NW
    steps = per_w // q
    assert steps % 2 == 0 and steps >= 4

    mesh = plsc.VectorSubcoreMesh(core_axis_name="c", subcore_axis_name="s")

    @functools.partial(
        pl.kernel,
        out_type=jax.ShapeDtypeStruct((n, 2), jnp.float32),
        mesh=mesh,
        scratch_types=[
            pltpu.VMEM((4 * q,), jnp.int32),
            pltpu.VMEM((4 * q,), jnp.int32),
            pltpu.VMEM((4 * q, 2), jnp.float32),
            pltpu.VMEM((4 * q, 2), jnp.float32),
            pltpu.SemaphoreType.DMA,
            pltpu.SemaphoreType.DMA,
        ],
        compiler_params=pltpu.CompilerParams(
            use_tc_tiling_on_sc=False, needs_layout_passes=False
        ),
    )
    def body(ids_hbm, tab_hbm, out_hbm, idx0, idx1, rows0, rows1, sem0, sem1):
        wid = lax.axis_index("s") * 2 + lax.axis_index("c")
        base = wid * per_w
        bufs = ((idx0, rows0, sem0), (idx1, rows1, sem1))

        def stage_and_fire(g, idx_v, rows_v, sem):
            pltpu.sync_copy(ids_hbm.at[pl.ds(base + g * q, q)], idx_v.at[pl.ds(0, q)])
            pltpu.async_copy(tab_hbm.at[idx_v], rows_v, sem)

        for b in range(2):
            stage_and_fire(b, *bufs[b])

        def pair(k, carry):
            for b in range(2):
                idx_v, rows_v, sem = bufs[b]
                g = 2 * k + b
                pltpu.make_async_copy(tab_hbm.at[idx_v], rows_v, sem).wait()
                pltpu.sync_copy(rows_v, out_hbm.at[pl.ds(base + g * q, 4 * q)])

                @pl.when(g + 2 < steps)
                def _():
                    stage_and_fire(g + 2, idx_v, rows_v, sem)

            return carry

        lax.fori_loop(0, steps // 2, pair, 0)

    return body(flat_ids, W)


def kernel(token_ids, W):
    b, h = token_ids.shape
    n = b * h
    flat = token_ids.reshape(n).astype(jnp.int32)
    out = _sc_gather(flat, W, n, 1024)
    return out.reshape(b, h, 2)
